# Initial kernel scaffold; baseline (speedup 1.0000x reference)
import jax
import jax.numpy as jnp
from jax.experimental import pallas as pl


def kernel(x, edge_index, batch, W1, b1, W2, b2, W3, b3, Wl, bl):
    raise NotImplementedError("write your pallas kernel here")


# speedup vs baseline: 7.3364x; 7.3364x over previous
import functools

import jax
import jax.numpy as jnp
from jax import lax
from jax.experimental import pallas as pl
from jax.experimental.pallas import tpu as pltpu
from jax.experimental.pallas import tpu_sc as plsc

NC = 2
NS = 16
CHUNK = 128


def _sc_mesh():
    return plsc.VectorSubcoreMesh(
        core_axis_name="c", subcore_axis_name="s", num_cores=NC,
        num_subcores=NS)


def _deg_kernel(n, e_pad, n_acc, z_rows):
    per_tile = e_pad // (NC * NS)
    n_ch = per_tile // CHUNK

    @functools.partial(
        pl.kernel,
        out_type=jax.ShapeDtypeStruct((NC, n_acc, 128), jnp.float32),
        mesh=_sc_mesh(),
        scratch_types=[
            pltpu.VMEM_SHARED((n_acc, 128), jnp.float32),
            pltpu.VMEM((CHUNK, 128), jnp.float32),
            pltpu.VMEM((CHUNK,), jnp.int32),
        ],
    )
    def k(dst_hbm, ones_hbm, zeros_hbm, out_hbm, acc, ones_v, idx_v):
        c = lax.axis_index("c")
        s = lax.axis_index("s")
        tid = c * NS + s
        pltpu.sync_copy(zeros_hbm, acc.at[pl.ds(s * z_rows, z_rows)])
        pltpu.sync_copy(ones_hbm, ones_v)
        plsc.subcore_barrier()
        base = tid * per_tile

        def step(i, carry):
            pltpu.sync_copy(dst_hbm.at[pl.ds(base + i * CHUNK, CHUNK)], idx_v)
            pltpu.sync_copy(ones_v, acc.at[idx_v], add=True)
            return carry

        lax.fori_loop(0, n_ch, step, 0)
        plsc.subcore_barrier()
        pltpu.sync_copy(acc.at[pl.ds(s * z_rows, z_rows)],
                        out_hbm.at[c, pl.ds(s * z_rows, z_rows)])

    return k


def _scatter_kernel(n, e_pad, n_acc, z_rows):
    per_tile = e_pad // NS
    n_ch = per_tile // CHUNK

    @functools.partial(
        pl.kernel,
        out_type=jax.ShapeDtypeStruct((NC, n_acc, 128), jnp.float32),
        mesh=_sc_mesh(),
        scratch_types=[
            pltpu.VMEM_SHARED((n_acc, 128), jnp.float32),
            pltpu.VMEM((CHUNK, 128), jnp.float32),
            pltpu.VMEM((CHUNK,), jnp.int32),
            pltpu.VMEM((CHUNK,), jnp.int32),
            pltpu.SemaphoreType.DMA,
        ],
    )
    def k(hp_hbm, srcs_hbm, dst_hbm, zeros_hbm, out_hbm,
          acc, rows_v, sidx_v, didx_v, sem):
        c = lax.axis_index("c")
        s = lax.axis_index("s")
        pltpu.sync_copy(zeros_hbm, acc.at[pl.ds(s * z_rows, z_rows)])
        plsc.subcore_barrier()
        base = s * per_tile

        def step(i, carry):
            e0 = base + i * CHUNK
            pltpu.sync_copy(srcs_hbm.at[c, pl.ds(e0, CHUNK)], sidx_v)
            pltpu.sync_copy(dst_hbm.at[pl.ds(e0, CHUNK)], didx_v)
            pltpu.async_copy(hp_hbm.at[sidx_v], rows_v, sem).wait()
            pltpu.sync_copy(rows_v, acc.at[didx_v], add=True)
            return carry

        lax.fori_loop(0, n_ch, step, 0)
        plsc.subcore_barrier()
        pltpu.sync_copy(acc.at[pl.ds(s * z_rows, z_rows)],
                        out_hbm.at[c, pl.ds(s * z_rows, z_rows)])

    return k


def _tc0(cnt, x, w1r, n_blk, blk):
    n = x.shape[0]

    def body(cnt_ref, x_ref, w_ref, hp_ref, dinv_ref):
        deg = cnt_ref[0, :, 0:1] + cnt_ref[1, :, 0:1] + 1.0
        dinv = lax.rsqrt(deg)
        dinv_ref[...] = dinv
        h = jnp.dot(x_ref[...], w_ref[0], preferred_element_type=jnp.float32)
        hp_ref[...] = (dinv * h)[None]

    return pl.pallas_call(
        body,
        grid=(n_blk, NC),
        in_specs=[
            pl.BlockSpec((NC, blk, 128), lambda i, j: (0, i, 0)),
            pl.BlockSpec((blk, 128), lambda i, j: (i, 0)),
            pl.BlockSpec((1, 128, 128), lambda i, j: (j, 0, 0)),
        ],
        out_specs=[
            pl.BlockSpec((1, blk, 128), lambda i, j: (j, i, 0)),
            pl.BlockSpec((blk, 1), lambda i, j: (i, 0)),
        ],
        out_shape=[
            jax.ShapeDtypeStruct((NC, n, 128), jnp.float32),
            jax.ShapeDtypeStruct((n, 1), jnp.float32),
        ],
    )(cnt, x, w1r)


def _tc_mid(acc, hp, dinv, b2, wr, n_blk, blk):
    n = hp.shape[1]

    def body(acc_ref, hp_ref, dinv_ref, b_ref, w_ref, out_ref):
        dinv = dinv_ref[...]
        z0 = jnp.maximum(dinv * (acc_ref[0] + hp_ref[0]) + b_ref[0:1, :], 0.0)
        z1 = jnp.maximum(dinv * (acc_ref[1] + hp_ref[1]) + b_ref[1:2, :], 0.0)
        h = (jnp.dot(z0, w_ref[0, 0], preferred_element_type=jnp.float32)
             + jnp.dot(z1, w_ref[1, 0], preferred_element_type=jnp.float32))
        out_ref[...] = (dinv * h)[None]

    return pl.pallas_call(
        body,
        grid=(n_blk, NC),
        in_specs=[
            pl.BlockSpec((NC, blk, 128), lambda i, j: (0, i, 0)),
            pl.BlockSpec((NC, blk, 128), lambda i, j: (0, i, 0)),
            pl.BlockSpec((blk, 1), lambda i, j: (i, 0)),
            pl.BlockSpec((NC, 128), lambda i, j: (0, 0)),
            pl.BlockSpec((NC, 1, 128, 128), lambda i, j: (0, j, 0, 0)),
        ],
        out_specs=pl.BlockSpec((1, blk, 128), lambda i, j: (j, i, 0)),
        out_shape=jax.ShapeDtypeStruct((NC, n, 128), jnp.float32),
    )(acc, hp, dinv, b2, wr)


def _tc_fin(acc, hp, dinv, b2, batch3, wlp, blp, n_blk, blk, g):
    def body(acc_ref, hp_ref, dinv_ref, b_ref, bt_ref, wl_ref, bl_ref,
             out_ref, pooled, cnts):
        i = pl.program_id(0)

        @pl.when(i == 0)
        def _():
            pooled[...] = jnp.zeros_like(pooled)
            cnts[...] = jnp.zeros_like(cnts)

        dinv = dinv_ref[...]
        z0 = dinv * (acc_ref[0] + hp_ref[0]) + b_ref[0:1, :]
        z1 = dinv * (acc_ref[1] + hp_ref[1]) + b_ref[1:2, :]
        bt = bt_ref[0]
        gid = lax.broadcasted_iota(jnp.int32, (g, blk), 0)
        oh = (jnp.broadcast_to(bt, (g, blk)) == gid).astype(jnp.float32)
        pooled[:, 0:128] += jnp.dot(oh, z0, preferred_element_type=jnp.float32)
        pooled[:, 128:256] += jnp.dot(oh, z1,
                                      preferred_element_type=jnp.float32)
        cnts[...] += jnp.broadcast_to(jnp.sum(oh, axis=1, keepdims=True),
                                      cnts.shape)

        @pl.when(i == n_blk - 1)
        def _():
            cnt = jnp.maximum(cnts[:, 0:1], 1.0)
            pk = pooled[...] / cnt
            out_ref[...] = (jnp.dot(pk, wl_ref[...],
                                    preferred_element_type=jnp.float32)
                            + bl_ref[...])

    return pl.pallas_call(
        body,
        grid=(n_blk,),
        in_specs=[
            pl.BlockSpec((NC, blk, 128), lambda i: (0, i, 0)),
            pl.BlockSpec((NC, blk, 128), lambda i: (0, i, 0)),
            pl.BlockSpec((blk, 1), lambda i: (i, 0)),
            pl.BlockSpec((NC, 128), lambda i: (0, 0)),
            pl.BlockSpec((1, 1, blk), lambda i: (i, 0, 0)),
            pl.BlockSpec((256, 128), lambda i: (0, 0)),
            pl.BlockSpec((1, 128), lambda i: (0, 0)),
        ],
        out_specs=pl.BlockSpec((g, 128), lambda i: (0, 0)),
        out_shape=jax.ShapeDtypeStruct((g, 128), jnp.float32),
        scratch_shapes=[
            pltpu.VMEM((g, 256), jnp.float32),
            pltpu.VMEM((g, 128), jnp.float32),
        ],
    )(acc, hp, dinv, b2, batch3, wlp, blp)


def kernel(x, edge_index, batch, W1, b1, W2, b2, W3, b3, Wl, bl):
    n, f_in = x.shape
    e = edge_index.shape[1]
    h = W2.shape[0]
    c_out = Wl.shape[1]
    g = 64

    unit = NC * NS * CHUNK
    e_pad = ((e + unit - 1) // unit) * unit
    z_rows = ((n // NS) // 8 + 1) * 8
    np_ = NS * z_rows
    dummy = n

    src = edge_index[0]
    dst = edge_index[1]
    pad = e_pad - e
    srcp = jnp.concatenate([src, jnp.zeros((pad,), jnp.int32)])
    dstp = jnp.concatenate([dst, jnp.full((pad,), dummy, jnp.int32)])
    srcs2 = jnp.stack([srcp, srcp + np_])

    ones128 = jnp.ones((CHUNK, 128), jnp.float32)
    zeros128 = jnp.zeros((z_rows, 128), jnp.float32)

    xp = jnp.pad(x, ((0, np_ - n), (0, 0)))
    batchp = jnp.concatenate([batch, jnp.full((np_ - n,), g, jnp.int32)])

    w1r = W1.reshape(f_in, NC, 128).transpose(1, 0, 2)
    w2r = W2.reshape(NC, 128, NC, 128).transpose(0, 2, 1, 3)
    w3r = W3.reshape(NC, 128, NC, 128).transpose(0, 2, 1, 3)
    b1_2 = b1.reshape(NC, 128)
    b2_2 = b2.reshape(NC, 128)
    b3_2 = b3.reshape(NC, 128)
    wlp = jnp.pad(Wl, ((0, 0), (0, 128 - c_out)))
    blp = jnp.pad(bl, (0, 128 - c_out))[None, :]

    blk = 2 * z_rows
    n_blk = np_ // blk
    batch3 = batchp.reshape(n_blk, 1, blk)

    cnt = _deg_kernel(n, e_pad, np_, z_rows)(dstp, ones128, zeros128)
    hp1, dinv = _tc0(cnt, xp, w1r, n_blk, blk)

    scat = _scatter_kernel(n, e_pad, np_, z_rows)
    acc1 = scat(hp1.reshape(NC * np_, 128), srcs2, dstp, zeros128)
    hp2 = _tc_mid(acc1, hp1, dinv, b1_2, w2r, n_blk, blk)
    acc2 = scat(hp2.reshape(NC * np_, 128), srcs2, dstp, zeros128)
    hp3 = _tc_mid(acc2, hp2, dinv, b2_2, w3r, n_blk, blk)
    acc3 = scat(hp3.reshape(NC * np_, 128), srcs2, dstp, zeros128)

    outp = _tc_fin(acc3, hp3, dinv, b3_2, batch3, wlp, blp, n_blk, blk, g)
    return outp[:, :c_out]


# speedup vs baseline: 9.9259x; 1.3530x over previous
import functools

import jax
import jax.numpy as jnp
from jax import lax
from jax.experimental import pallas as pl
from jax.experimental.pallas import tpu as pltpu
from jax.experimental.pallas import tpu_sc as plsc

NC = 2
NS = 16
CHUNK = 128
NBUF = 2


def _sc_mesh():
    return plsc.VectorSubcoreMesh(
        core_axis_name="c", subcore_axis_name="s", num_cores=NC,
        num_subcores=NS)


def _deg_kernel(n, e_pad, n_acc, z_rows):
    per_tile = e_pad // (NC * NS)
    n_ch = per_tile // CHUNK
    n_z = z_rows // CHUNK

    @functools.partial(
        pl.kernel,
        out_type=jax.ShapeDtypeStruct((NC, n_acc, 128), jnp.float32),
        mesh=_sc_mesh(),
        scratch_types=[
            pltpu.VMEM_SHARED((n_acc, 128), jnp.float32),
            pltpu.VMEM((CHUNK, 128), jnp.float32),
            pltpu.VMEM((CHUNK,), jnp.int32),
        ],
    )
    def k(dst_hbm, ones_hbm, zeros_hbm, out_hbm, acc, ones_v, idx_v):
        c = lax.axis_index("c")
        s = lax.axis_index("s")
        tid = c * NS + s
        pltpu.sync_copy(zeros_hbm, ones_v)
        for q in range(n_z):
            pltpu.sync_copy(ones_v,
                            acc.at[pl.ds(s * z_rows + q * CHUNK, CHUNK)])
        pltpu.sync_copy(ones_hbm, ones_v)
        plsc.subcore_barrier()
        base = tid * per_tile

        def step(i, carry):
            pltpu.sync_copy(dst_hbm.at[pl.ds(base + i * CHUNK, CHUNK)], idx_v)
            pltpu.sync_copy(ones_v, acc.at[idx_v], add=True)
            return carry

        lax.fori_loop(0, n_ch, step, 0)
        plsc.subcore_barrier()
        for q in range(n_z):
            row0 = s * z_rows + q * CHUNK
            pltpu.sync_copy(acc.at[pl.ds(row0, CHUNK)], ones_v)
            pltpu.sync_copy(ones_v, out_hbm.at[c, pl.ds(row0, CHUNK)])

    return k


def _scatter_kernel(n, e_pad, n_acc, z_rows):
    ept = e_pad // NS
    n_ch = ept // CHUNK
    n_rounds = n_ch // NBUF
    n_z = z_rows // CHUNK

    @functools.partial(
        pl.kernel,
        out_type=jax.ShapeDtypeStruct((NC, n_acc, 128), jnp.float32),
        mesh=_sc_mesh(),
        scratch_types=[
            pltpu.VMEM_SHARED((n_acc, 128), jnp.float32),
            pltpu.VMEM((CHUNK,), jnp.int32),
            pltpu.VMEM((CHUNK,), jnp.int32),
            pltpu.VMEM((CHUNK,), jnp.int32),
            pltpu.VMEM((CHUNK,), jnp.int32),
            pltpu.VMEM((CHUNK, 128), jnp.float32),
            pltpu.VMEM((CHUNK, 128), jnp.float32),
            pltpu.SemaphoreType.DMA,
            pltpu.SemaphoreType.DMA,
            pltpu.SemaphoreType.DMA,
            pltpu.SemaphoreType.DMA,
        ],
    )
    def k(hp_hbm, srcs_hbm, dst_hbm, zeros_hbm, out_hbm,
          acc, sidx0, sidx1, didx0, didx1, g0, g1, gs0, gs1, ss0, ss1):
        c = lax.axis_index("c")
        s = lax.axis_index("s")
        bufs = ((g0, sidx0, didx0, gs0, ss0), (g1, sidx1, didx1, gs1, ss1))
        pltpu.sync_copy(zeros_hbm, g0)
        for q in range(n_z):
            pltpu.sync_copy(g0, acc.at[pl.ds(s * z_rows + q * CHUNK, CHUNK)])
        plsc.subcore_barrier()
        base = s * ept

        def round_body(r, carry):
            for b, (gb, sb, db, gsm, ssm) in enumerate(bufs):
                off_cur = (r * NBUF + b) * CHUNK
                off_prev = off_cur - NBUF * CHUNK

                @pl.when(r > 0)
                def _():
                    pltpu.sync_copy(
                        dst_hbm.at[pl.ds(base + off_prev, CHUNK)], db)
                    pltpu.make_async_copy(hp_hbm.at[sb], gb, gsm).wait()
                    pltpu.async_copy(gb, acc.at[db], ssm, add=True)
                    pltpu.make_async_copy(gb, acc.at[db], ssm).wait()

                @pl.when(r < n_rounds)
                def _():
                    pltpu.sync_copy(
                        srcs_hbm.at[pl.ds(c * e_pad + base + off_cur, CHUNK)],
                        sb)
                    pltpu.async_copy(hp_hbm.at[sb], gb, gsm)
            return carry

        lax.fori_loop(0, n_rounds + 1, round_body, 0)
        plsc.subcore_barrier()
        for q in range(n_z):
            row0 = s * z_rows + q * CHUNK
            pltpu.sync_copy(acc.at[pl.ds(row0, CHUNK)], g0)
            pltpu.sync_copy(g0, out_hbm.at[c, pl.ds(row0, CHUNK)])

    return k


def _tc0(cnt, x, w1r, n_blk, blk):
    n = x.shape[0]

    def body(cnt_ref, x_ref, w_ref, hp_ref, dinv_ref):
        deg = cnt_ref[0, :, 0:1] + cnt_ref[1, :, 0:1] + 1.0
        dinv = lax.rsqrt(deg)
        dinv_ref[...] = dinv
        h = jnp.dot(x_ref[...], w_ref[0], preferred_element_type=jnp.float32)
        hp_ref[...] = (dinv * h)[None]

    return pl.pallas_call(
        body,
        grid=(n_blk, NC),
        in_specs=[
            pl.BlockSpec((NC, blk, 128), lambda i, j: (0, i, 0)),
            pl.BlockSpec((blk, 128), lambda i, j: (i, 0)),
            pl.BlockSpec((1, 128, 128), lambda i, j: (j, 0, 0)),
        ],
        out_specs=[
            pl.BlockSpec((1, blk, 128), lambda i, j: (j, i, 0)),
            pl.BlockSpec((blk, 1), lambda i, j: (i, 0)),
        ],
        out_shape=[
            jax.ShapeDtypeStruct((NC, n, 128), jnp.float32),
            jax.ShapeDtypeStruct((n, 1), jnp.float32),
        ],
    )(cnt, x, w1r)


def _tc_mid(acc, hp, dinv, b2, wr, n_blk, blk):
    n = hp.shape[1]

    def body(acc_ref, hp_ref, dinv_ref, b_ref, w_ref, out_ref):
        dinv = dinv_ref[...]
        z0 = jnp.maximum(dinv * (acc_ref[0] + hp_ref[0]) + b_ref[0:1, :], 0.0)
        z1 = jnp.maximum(dinv * (acc_ref[1] + hp_ref[1]) + b_ref[1:2, :], 0.0)
        h = (jnp.dot(z0, w_ref[0, 0], preferred_element_type=jnp.float32)
             + jnp.dot(z1, w_ref[1, 0], preferred_element_type=jnp.float32))
        out_ref[...] = (dinv * h)[None]

    return pl.pallas_call(
        body,
        grid=(n_blk, NC),
        in_specs=[
            pl.BlockSpec((NC, blk, 128), lambda i, j: (0, i, 0)),
            pl.BlockSpec((NC, blk, 128), lambda i, j: (0, i, 0)),
            pl.BlockSpec((blk, 1), lambda i, j: (i, 0)),
            pl.BlockSpec((NC, 128), lambda i, j: (0, 0)),
            pl.BlockSpec((NC, 1, 128, 128), lambda i, j: (0, j, 0, 0)),
        ],
        out_specs=pl.BlockSpec((1, blk, 128), lambda i, j: (j, i, 0)),
        out_shape=jax.ShapeDtypeStruct((NC, n, 128), jnp.float32),
    )(acc, hp, dinv, b2, wr)


def _tc_fin(acc, hp, dinv, b2, batch3, wlp, blp, n_blk, blk, g):
    def body(acc_ref, hp_ref, dinv_ref, b_ref, bt_ref, wl_ref, bl_ref,
             out_ref, pooled, cnts):
        i = pl.program_id(0)

        @pl.when(i == 0)
        def _():
            pooled[...] = jnp.zeros_like(pooled)
            cnts[...] = jnp.zeros_like(cnts)

        dinv = dinv_ref[...]
        z0 = dinv * (acc_ref[0] + hp_ref[0]) + b_ref[0:1, :]
        z1 = dinv * (acc_ref[1] + hp_ref[1]) + b_ref[1:2, :]
        bt = bt_ref[0]
        gid = lax.broadcasted_iota(jnp.int32, (g, blk), 0)
        oh = (jnp.broadcast_to(bt, (g, blk)) == gid).astype(jnp.float32)
        pooled[:, 0:128] += jnp.dot(oh, z0, preferred_element_type=jnp.float32)
        pooled[:, 128:256] += jnp.dot(oh, z1,
                                      preferred_element_type=jnp.float32)
        cnts[...] += jnp.broadcast_to(jnp.sum(oh, axis=1, keepdims=True),
                                      cnts.shape)

        @pl.when(i == n_blk - 1)
        def _():
            cnt = jnp.maximum(cnts[:, 0:1], 1.0)
            pk = pooled[...] / cnt
            out_ref[...] = (jnp.dot(pk, wl_ref[...],
                                    preferred_element_type=jnp.float32)
                            + bl_ref[...])

    return pl.pallas_call(
        body,
        grid=(n_blk,),
        in_specs=[
            pl.BlockSpec((NC, blk, 128), lambda i: (0, i, 0)),
            pl.BlockSpec((NC, blk, 128), lambda i: (0, i, 0)),
            pl.BlockSpec((blk, 1), lambda i: (i, 0)),
            pl.BlockSpec((NC, 128), lambda i: (0, 0)),
            pl.BlockSpec((1, 1, blk), lambda i: (i, 0, 0)),
            pl.BlockSpec((256, 128), lambda i: (0, 0)),
            pl.BlockSpec((1, 128), lambda i: (0, 0)),
        ],
        out_specs=pl.BlockSpec((g, 128), lambda i: (0, 0)),
        out_shape=jax.ShapeDtypeStruct((g, 128), jnp.float32),
        scratch_shapes=[
            pltpu.VMEM((g, 256), jnp.float32),
            pltpu.VMEM((g, 128), jnp.float32),
        ],
    )(acc, hp, dinv, b2, batch3, wlp, blp)


def kernel(x, edge_index, batch, W1, b1, W2, b2, W3, b3, Wl, bl):
    n, f_in = x.shape
    e = edge_index.shape[1]
    h = W2.shape[0]
    c_out = Wl.shape[1]
    g = 64

    unit = NS * CHUNK * NBUF
    e_pad = ((e + unit - 1) // unit) * unit
    z_rows = ((n // NS) // CHUNK + 1) * CHUNK
    np_ = NS * z_rows
    dummy = n

    src = edge_index[0]
    dst = edge_index[1]
    pad = e_pad - e
    srcp = jnp.concatenate([src, jnp.zeros((pad,), jnp.int32)])
    dstp = jnp.concatenate([dst, jnp.full((pad,), dummy, jnp.int32)])
    srcs2 = jnp.concatenate([srcp, srcp + np_])

    ones128 = jnp.ones((CHUNK, 128), jnp.float32)
    zeros128 = jnp.zeros((CHUNK, 128), jnp.float32)

    xp = jnp.pad(x, ((0, np_ - n), (0, 0)))
    batchp = jnp.concatenate([batch, jnp.full((np_ - n,), g, jnp.int32)])

    w1r = W1.reshape(f_in, NC, 128).transpose(1, 0, 2)
    w2r = W2.reshape(NC, 128, NC, 128).transpose(0, 2, 1, 3)
    w3r = W3.reshape(NC, 128, NC, 128).transpose(0, 2, 1, 3)
    b1_2 = b1.reshape(NC, 128)
    b2_2 = b2.reshape(NC, 128)
    b3_2 = b3.reshape(NC, 128)
    wlp = jnp.pad(Wl, ((0, 0), (0, 128 - c_out)))
    blp = jnp.pad(bl, (0, 128 - c_out))[None, :]

    blk = 2 * z_rows
    n_blk = np_ // blk
    batch3 = batchp.reshape(n_blk, 1, blk)

    cnt = _deg_kernel(n, e_pad, np_, z_rows)(dstp, ones128, zeros128)
    hp1, dinv = _tc0(cnt, xp, w1r, n_blk, blk)

    scat = _scatter_kernel(n, e_pad, np_, z_rows)
    acc1 = scat(hp1.reshape(NC * np_, 128), srcs2, dstp, zeros128)
    hp2 = _tc_mid(acc1, hp1, dinv, b1_2, w2r, n_blk, blk)
    acc2 = scat(hp2.reshape(NC * np_, 128), srcs2, dstp, zeros128)
    hp3 = _tc_mid(acc2, hp2, dinv, b2_2, w3r, n_blk, blk)
    acc3 = scat(hp3.reshape(NC * np_, 128), srcs2, dstp, zeros128)

    outp = _tc_fin(acc3, hp3, dinv, b3_2, batch3, wlp, blp, n_blk, blk, g)
    return outp[:, :c_out]
